# baseline (speedup 1.0000x reference)
import jax
import jax.numpy as jnp
from jax import lax
from jax.experimental import pallas as pl
from jax.experimental.pallas import tpu as pltpu
from jax.experimental.pallas import tpu_sc as plsc

NUM_EXPERTS = 8
TOP_K = 2
HIDDEN = 2048
TOKENS = 2048
ROWS = NUM_EXPERTS * TOKENS


MM_GRID = 8
MM_TBLK = TOKENS // MM_GRID


def _mm_body(w_ref, h_ref, scores_ref):
    lt = jax.lax.dot_general(
        w_ref[...], h_ref[...], (((1,), (1,)), ((), ())),
        preferred_element_type=jnp.float32)
    eidx = jax.lax.broadcasted_iota(jnp.int32, lt.shape, 0)
    m1 = jnp.max(lt, axis=0, keepdims=True)
    i1 = jnp.min(jnp.where(lt == m1, eidx, NUM_EXPERTS), axis=0, keepdims=True)
    masked = jnp.where(eidx == i1, -jnp.inf, lt)
    m2 = jnp.max(masked, axis=0, keepdims=True)
    i2 = jnp.min(jnp.where(masked == m2, eidx, NUM_EXPERTS), axis=0,
                 keepdims=True)
    keep = (eidx == i1) | (eidx == i2)
    scores_ref[...] = jnp.where(keep, jax.nn.sigmoid(lt), 0.0)


def _tc_scores(hidden_states, W):
    return pl.pallas_call(
        _mm_body,
        grid=(MM_GRID,),
        in_specs=[
            pl.BlockSpec((NUM_EXPERTS, HIDDEN), lambda i: (0, 0)),
            pl.BlockSpec((MM_TBLK, HIDDEN), lambda i: (i, 0)),
        ],
        out_specs=pl.BlockSpec((NUM_EXPERTS, MM_TBLK), lambda i: (0, i)),
        out_shape=jax.ShapeDtypeStruct((NUM_EXPERTS, TOKENS), jnp.float32),
    )(W, hidden_states)


_SC_INFO = plsc.get_sparse_core_info()
_NC, _NS, _L = _SC_INFO.num_cores, _SC_INFO.num_subcores, _SC_INFO.num_lanes
_NW = _NC * _NS
_NPER = ROWS // TOKENS
_VPW = TOKENS // _NW
_CROWS = 16
_NCHUNK = _VPW // _CROWS


def _fill(buf, vbase):
    def body(j, _):
        val = jnp.zeros((_L,), jnp.int32) + (vbase + j)
        for c in range(HIDDEN // _L):
            buf[j, pl.ds(c * _L, _L)] = val
        return ()

    lax.fori_loop(0, _CROWS, body, ())


def _sc_idx_body(out_hbm, buf_a, buf_b, sem_a, sem_b):
    wid = lax.axis_index("s") * _NC + lax.axis_index("c")
    vbase = wid * _VPW
    bufs = (buf_a, buf_b)
    sems = (sem_a, sem_b)
    pending = [None, None]
    for chunk in range(_NCHUNK):
        slot = chunk % 2
        if pending[slot] is not None:
            for cp in pending[slot]:
                cp.wait()
        base = vbase + chunk * _CROWS
        _fill(bufs[slot], base)
        copies = []
        for p in range(_NPER):
            copies.append(pltpu.async_copy(
                bufs[slot],
                out_hbm.at[pl.ds(p * TOKENS + base, _CROWS), :],
                sems[slot]))
        pending[slot] = copies
    for cps in pending:
        if cps is not None:
            for cp in cps:
                cp.wait()


_sc_indices = pl.kernel(
    _sc_idx_body,
    mesh=plsc.VectorSubcoreMesh(core_axis_name="c", subcore_axis_name="s"),
    out_type=jax.ShapeDtypeStruct((ROWS, HIDDEN), jnp.int32),
    scratch_types=[
        pltpu.VMEM((_CROWS, HIDDEN), jnp.int32),
        pltpu.VMEM((_CROWS, HIDDEN), jnp.int32),
        pltpu.SemaphoreType.DMA,
        pltpu.SemaphoreType.DMA,
    ],
)


def kernel(hidden_states, W):
    indices = _sc_indices()
    scores = _tc_scores(hidden_states, W)
    probs = scores.reshape(-1, 1)
    return (scores, indices, probs)


# speedup vs baseline: 1.3216x; 1.3216x over previous
import jax
import jax.numpy as jnp
from jax.experimental import pallas as pl

NUM_EXPERTS = 8
TOP_K = 2
HIDDEN = 2048
TOKENS = 2048
ROWS = NUM_EXPERTS * TOKENS

GRID = 8
TBLK = TOKENS // GRID
RBLK = ROWS // GRID


def _body(w_ref, h_ref, scores_ref, idx_ref):
    i = pl.program_id(0)
    lt = jax.lax.dot_general(
        w_ref[...], h_ref[...], (((1,), (1,)), ((), ())),
        preferred_element_type=jnp.float32)
    eidx = jax.lax.broadcasted_iota(jnp.int32, lt.shape, 0)
    m1 = jnp.max(lt, axis=0, keepdims=True)
    i1 = jnp.min(jnp.where(lt == m1, eidx, NUM_EXPERTS), axis=0, keepdims=True)
    masked = jnp.where(eidx == i1, -jnp.inf, lt)
    m2 = jnp.max(masked, axis=0, keepdims=True)
    i2 = jnp.min(jnp.where(masked == m2, eidx, NUM_EXPERTS), axis=0,
                 keepdims=True)
    keep = (eidx == i1) | (eidx == i2)
    scores_ref[...] = jnp.where(keep, jax.nn.sigmoid(lt), 0.0)
    ridx = jax.lax.broadcasted_iota(jnp.int32, (RBLK, HIDDEN), 0)
    idx_ref[...] = (i * RBLK) % TOKENS + ridx


def kernel(hidden_states, W):
    scores, indices = pl.pallas_call(
        _body,
        grid=(GRID,),
        in_specs=[
            pl.BlockSpec((NUM_EXPERTS, HIDDEN), lambda i: (0, 0)),
            pl.BlockSpec((TBLK, HIDDEN), lambda i: (i, 0)),
        ],
        out_specs=[
            pl.BlockSpec((NUM_EXPERTS, TBLK), lambda i: (0, i)),
            pl.BlockSpec((RBLK, HIDDEN), lambda i: (i, 0)),
        ],
        out_shape=[
            jax.ShapeDtypeStruct((NUM_EXPERTS, TOKENS), jnp.float32),
            jax.ShapeDtypeStruct((ROWS, HIDDEN), jnp.int32),
        ],
    )(W, hidden_states)
    probs = scores.reshape(-1, 1)
    return (scores, indices, probs)
